# baseline (speedup 1.0000x reference)
import functools

import jax
import jax.numpy as jnp
from jax import lax
from jax.experimental import pallas as pl
from jax.experimental.pallas import tpu as pltpu
from jax.experimental.pallas import tpu_sc as plsc

B, L, D = 4096, 200, 128
N = B * L
NC, NS = 2, 16
NW = NC * NS
PER_W = N // NW
C = 256
NCHUNK = PER_W // C
NBUF = 3
NG = C // 16
NIB = C // 128
NP = 8 * 64
NJ = D // 16
PROWS = NP // NS


def _y_group(yiv, ytab_v, obv, coff):
    def ld_row(yb16, r):
        yb = jnp.full((16,), yb16[r], jnp.int32)
        return [plsc.load_gather(ytab_v, [yb + coff[j]]) for j in range(NJ)]

    def group(g, c2):
        yb16 = yiv[pl.ds(g * 16, 16)] * 128
        prev = ld_row(yb16, 0)
        for r in range(1, 16):
            yb = jnp.full((16,), yb16[r], jnp.int32)
            cur = []
            for j in range(NJ):
                cur.append(plsc.load_gather(ytab_v, [yb + coff[j]]))
                plsc.addupdate(obv.at[g * 16 + r - 1, pl.ds(j * 16, 16)],
                               prev[j])
            prev = cur
        for j in range(NJ):
            plsc.addupdate(obv.at[g * 16 + 15, pl.ds(j * 16, 16)], prev[j])
        return c2

    return group


def _sc_body(at_hbm, xi_hbm, yi_hbm, atab_hbm, xtab_hbm, ytab_hbm, out_hbm,
             ptab_sp, ytab_v, stage_v, stage2_v,
             ai0, ai1, ai2, xi0, xi1, xi2, yi0, yi1, yi2,
             pi0, pi1, pi2,
             ob0, ob1, ob2,
             si0, si1, si2, sg0, sg1, sg2, so0, so1, so2):
    wid = lax.axis_index("s") * NC + lax.axis_index("c")
    sid = lax.axis_index("s")
    base = wid * PER_W
    ai = (ai0, ai1, ai2)
    xi = (xi0, xi1, xi2)
    yi = (yi0, yi1, yi2)
    pi = (pi0, pi1, pi2)
    ob = (ob0, ob1, ob2)
    s_in = (si0, si1, si2)
    s_g = (sg0, sg1, sg2)
    s_out = (so0, so1, so2)

    pltpu.sync_copy(ytab_hbm, ytab_v)

    pltpu.sync_copy(atab_hbm, stage_v.at[pl.ds(0, 8 * D)])
    pltpu.sync_copy(xtab_hbm, stage_v.at[pl.ds(8 * D, 64 * D)])
    p0 = sid * PROWS

    def build_pair(k, c2):
        p = p0 + k
        a_off = (p >> 6) * D
        x_off = 8 * D + (p & 63) * D
        for j in range(NJ):
            av = stage_v[pl.ds(a_off + j * 16, 16)]
            xv = stage_v[pl.ds(x_off + j * 16, 16)]
            stage2_v[k, pl.ds(j * 16, 16)] = av + xv
        return c2

    lax.fori_loop(0, PROWS, build_pair, 0, unroll=False)
    pltpu.sync_copy(stage2_v, ptab_sp.at[pl.ds(p0, PROWS)])
    plsc.subcore_barrier()

    iota = lax.iota(jnp.int32, 16)
    coff = [iota + 16 * j for j in range(NJ)]

    def issue_idx(ci, b):
        off = base + ci * C
        pltpu.async_copy(at_hbm.at[pl.ds(off, C)], ai[b], s_in[b])
        pltpu.async_copy(xi_hbm.at[pl.ds(off, C)], xi[b], s_in[b])
        pltpu.async_copy(yi_hbm.at[pl.ds(off, C)], yi[b], s_in[b])

    def wait_idx(ci, b):
        off = base + ci * C
        pltpu.make_async_copy(at_hbm.at[pl.ds(off, C)], ai[b], s_in[b]).wait()
        pltpu.make_async_copy(xi_hbm.at[pl.ds(off, C)], xi[b], s_in[b]).wait()
        pltpu.make_async_copy(yi_hbm.at[pl.ds(off, C)], yi[b], s_in[b]).wait()

    def start_gather(b, drain):
        for q in range(NIB):
            for g in range(128 // 16):
                s16 = pl.ds(q * 128 + g * 16, 16)
                pi[b][q, pl.ds(g * 16, 16)] = ai[b][s16] * 64 + xi[b][s16]
        if drain:
            pltpu.make_async_copy(
                ob[b], out_hbm.at[pl.ds(0, C)], s_out[b]).wait()
        for q in range(NIB):
            pltpu.async_copy(ptab_sp.at[pi[b].at[q]],
                             ob[b].at[pl.ds(q * 128, 128)], s_g[b])

    def wait_gather(b, q):
        pltpu.make_async_copy(ptab_sp.at[pi[b].at[q]],
                              ob[b].at[pl.ds(q * 128, 128)],
                              s_g[b]).wait()

    def finish_chunk(ci, b):
        off = base + ci * C
        grp = _y_group(yi[b], ytab_v, ob[b], coff)
        gpb = 128 // 16
        for q in range(NIB):
            wait_gather(b, q)
            lax.fori_loop(q * gpb, (q + 1) * gpb, grp, 0, unroll=False)

        @pl.when(ci + NBUF < NCHUNK)
        def _prefetch():
            issue_idx(ci + NBUF, b)

        pltpu.async_copy(ob[b], out_hbm.at[pl.ds(off, C)], s_out[b])

    for b in range(NBUF):
        issue_idx(b, b)
    wait_idx(0, 0)
    start_gather(0, drain=False)

    def outer(s, carry):
        for b in range(NBUF):
            ci = s * NBUF + b
            bn = (b + 1) % NBUF
            if b == NBUF - 1:
                @pl.when(ci + 1 < NCHUNK)
                def _ahead():
                    wait_idx(ci + 1, bn)
                    start_gather(bn, drain=True)
            else:
                @pl.when(jnp.logical_and(ci + 1 < NCHUNK, s > 0))
                def _ahead2():
                    wait_idx(ci + 1, bn)
                    start_gather(bn, drain=True)

                @pl.when(jnp.logical_and(ci + 1 < NCHUNK, s == 0))
                def _ahead3():
                    wait_idx(ci + 1, bn)
                    start_gather(bn, drain=False)
            finish_chunk(ci, b)
        return carry

    lax.fori_loop(0, NCHUNK // NBUF, outer, 0, unroll=False)

    for t in range((NCHUNK // NBUF) * NBUF, NCHUNK):
        b = t % NBUF
        if t + 1 < NCHUNK:
            wait_idx(t + 1, (b + 1) % NBUF)
            start_gather((b + 1) % NBUF, drain=True)
        finish_chunk(t, b)

    for b in range(min(NBUF, NCHUNK)):
        pltpu.make_async_copy(ob[b], out_hbm.at[pl.ds(0, C)], s_out[b]).wait()


def kernel(action_type, x, y, action_table, x_table, y_table):
    at = action_type.reshape(N).astype(jnp.int32)
    xi = x.reshape(N).astype(jnp.int32)
    yi = y.reshape(N).astype(jnp.int32)

    mesh = plsc.VectorSubcoreMesh(core_axis_name="c", subcore_axis_name="s")
    run = functools.partial(
        pl.kernel,
        mesh=mesh,
        compiler_params=pltpu.CompilerParams(needs_layout_passes=False),
        out_type=jax.ShapeDtypeStruct((N, D), jnp.float32),
        scratch_types=(
            [pltpu.VMEM_SHARED((NP, D), jnp.float32),
             pltpu.VMEM((64 * D,), jnp.float32),
             pltpu.VMEM((72 * D,), jnp.float32),
             pltpu.VMEM((PROWS, D), jnp.float32)]
            + [pltpu.VMEM((C,), jnp.int32) for _ in range(3 * NBUF)]
            + [pltpu.VMEM((NIB, 128), jnp.int32) for _ in range(NBUF)]
            + [pltpu.VMEM((C, D), jnp.float32) for _ in range(NBUF)]
            + [pltpu.SemaphoreType.DMA for _ in range(3 * NBUF)]
        ),
    )(_sc_body)
    out = run(at, xi, yi,
              action_table.reshape(8 * D),
              x_table.reshape(64 * D),
              y_table.reshape(64 * D))
    return out.reshape(B, L, D)


# speedup vs baseline: 1.2108x; 1.2108x over previous
import functools

import jax
import jax.numpy as jnp
from jax import lax
from jax.experimental import pallas as pl
from jax.experimental.pallas import tpu as pltpu
from jax.experimental.pallas import tpu_sc as plsc

B, L, D = 4096, 200, 128
N = B * L
NC, NS = 2, 16
NW = NC * NS
PER_W = N // NW
C = 256
NCHUNK = PER_W // C
NBUF = 3
NG = C // 16
NIB = C // 128
NP = 8 * 64
NJ = D // 16
PROWS = NP // NS


def _y_pass(yiv, ytab_v, obv, coff):
    def ld_row(yb16, r):
        yb = jnp.full((16,), yb16[r], jnp.int32)
        return [plsc.load_gather(ytab_v, [yb + coff[j]]) for j in range(NJ)]

    def group(g, c2):
        yb16 = yiv[pl.ds(g * 16, 16)] * 128
        prev = ld_row(yb16, 0)
        for r in range(1, 16):
            yb = jnp.full((16,), yb16[r], jnp.int32)
            cur = []
            for j in range(NJ):
                cur.append(plsc.load_gather(ytab_v, [yb + coff[j]]))
                plsc.addupdate(obv.at[g * 16 + r - 1, pl.ds(j * 16, 16)],
                               prev[j])
            prev = cur
        for j in range(NJ):
            plsc.addupdate(obv.at[g * 16 + 15, pl.ds(j * 16, 16)], prev[j])
        return c2

    lax.fori_loop(0, NG, group, 0, unroll=False)


def _sc_body(at_hbm, xi_hbm, yi_hbm, atab_hbm, xtab_hbm, ytab_hbm, out_hbm,
             ptab_sp, ytab_v, stage_v, stage2_v,
             ai0, ai1, ai2, xi0, xi1, xi2, yi0, yi1, yi2,
             pi0, pi1, pi2,
             ob0, ob1, ob2,
             si0, si1, si2, sg0, sg1, sg2, so0, so1, so2):
    wid = lax.axis_index("s") * NC + lax.axis_index("c")
    sid = lax.axis_index("s")
    base = wid * PER_W
    ai = (ai0, ai1, ai2)
    xi = (xi0, xi1, xi2)
    yi = (yi0, yi1, yi2)
    pi = (pi0, pi1, pi2)
    ob = (ob0, ob1, ob2)
    s_in = (si0, si1, si2)
    s_g = (sg0, sg1, sg2)
    s_out = (so0, so1, so2)

    pltpu.sync_copy(ytab_hbm, ytab_v)

    pltpu.sync_copy(atab_hbm, stage_v.at[pl.ds(0, 8 * D)])
    pltpu.sync_copy(xtab_hbm, stage_v.at[pl.ds(8 * D, 64 * D)])
    p0 = sid * PROWS

    def build_pair(k, c2):
        p = p0 + k
        a_off = (p >> 6) * D
        x_off = 8 * D + (p & 63) * D
        for j in range(NJ):
            av = stage_v[pl.ds(a_off + j * 16, 16)]
            xv = stage_v[pl.ds(x_off + j * 16, 16)]
            stage2_v[k, pl.ds(j * 16, 16)] = av + xv
        return c2

    lax.fori_loop(0, PROWS, build_pair, 0, unroll=False)
    pltpu.sync_copy(stage2_v, ptab_sp.at[pl.ds(p0, PROWS)])
    plsc.subcore_barrier()

    iota = lax.iota(jnp.int32, 16)
    coff = [iota + 16 * j for j in range(NJ)]

    def issue_idx(ci, b):
        off = base + ci * C
        pltpu.async_copy(at_hbm.at[pl.ds(off, C)], ai[b], s_in[b])
        pltpu.async_copy(xi_hbm.at[pl.ds(off, C)], xi[b], s_in[b])
        pltpu.async_copy(yi_hbm.at[pl.ds(off, C)], yi[b], s_in[b])

    def wait_idx(ci, b):
        off = base + ci * C
        pltpu.make_async_copy(at_hbm.at[pl.ds(off, C)], ai[b], s_in[b]).wait()
        pltpu.make_async_copy(xi_hbm.at[pl.ds(off, C)], xi[b], s_in[b]).wait()
        pltpu.make_async_copy(yi_hbm.at[pl.ds(off, C)], yi[b], s_in[b]).wait()

    def start_gather(b, drain):
        for q in range(NIB):
            for g in range(128 // 16):
                s16 = pl.ds(q * 128 + g * 16, 16)
                pi[b][q, pl.ds(g * 16, 16)] = ai[b][s16] * 64 + xi[b][s16]
        if drain:
            pltpu.make_async_copy(
                ob[b], out_hbm.at[pl.ds(0, C)], s_out[b]).wait()
        for q in range(NIB):
            pltpu.async_copy(ptab_sp.at[pi[b].at[q]],
                             ob[b].at[pl.ds(q * 128, 128)], s_g[b])

    def wait_gather(b):
        for q in range(NIB):
            pltpu.make_async_copy(ptab_sp.at[pi[b].at[q]],
                                  ob[b].at[pl.ds(q * 128, 128)],
                                  s_g[b]).wait()

    def finish_chunk(ci, b):
        off = base + ci * C
        wait_gather(b)
        _y_pass(yi[b], ytab_v, ob[b], coff)

        @pl.when(ci + NBUF < NCHUNK)
        def _prefetch():
            issue_idx(ci + NBUF, b)

        pltpu.async_copy(ob[b], out_hbm.at[pl.ds(off, C)], s_out[b])

    for b in range(NBUF):
        issue_idx(b, b)
    wait_idx(0, 0)
    start_gather(0, drain=False)

    def outer(s, carry):
        for b in range(NBUF):
            ci = s * NBUF + b
            bn = (b + 1) % NBUF
            if b == NBUF - 1:
                @pl.when(ci + 1 < NCHUNK)
                def _ahead():
                    wait_idx(ci + 1, bn)
                    start_gather(bn, drain=True)
            else:
                @pl.when(jnp.logical_and(ci + 1 < NCHUNK, s > 0))
                def _ahead2():
                    wait_idx(ci + 1, bn)
                    start_gather(bn, drain=True)

                @pl.when(jnp.logical_and(ci + 1 < NCHUNK, s == 0))
                def _ahead3():
                    wait_idx(ci + 1, bn)
                    start_gather(bn, drain=False)
            finish_chunk(ci, b)
        return carry

    lax.fori_loop(0, NCHUNK // NBUF, outer, 0, unroll=False)

    for t in range((NCHUNK // NBUF) * NBUF, NCHUNK):
        b = t % NBUF
        if t + 1 < NCHUNK:
            wait_idx(t + 1, (b + 1) % NBUF)
            start_gather((b + 1) % NBUF, drain=True)
        finish_chunk(t, b)

    for b in range(min(NBUF, NCHUNK)):
        pltpu.make_async_copy(ob[b], out_hbm.at[pl.ds(0, C)], s_out[b]).wait()


def kernel(action_type, x, y, action_table, x_table, y_table):
    at = action_type.reshape(N).astype(jnp.int32)
    xi = x.reshape(N).astype(jnp.int32)
    yi = y.reshape(N).astype(jnp.int32)

    mesh = plsc.VectorSubcoreMesh(core_axis_name="c", subcore_axis_name="s")
    run = functools.partial(
        pl.kernel,
        mesh=mesh,
        compiler_params=pltpu.CompilerParams(needs_layout_passes=False),
        out_type=jax.ShapeDtypeStruct((N, D), jnp.float32),
        scratch_types=(
            [pltpu.VMEM_SHARED((NP, D), jnp.float32),
             pltpu.VMEM((64 * D,), jnp.float32),
             pltpu.VMEM((72 * D,), jnp.float32),
             pltpu.VMEM((PROWS, D), jnp.float32)]
            + [pltpu.VMEM((C,), jnp.int32) for _ in range(3 * NBUF)]
            + [pltpu.VMEM((NIB, 128), jnp.int32) for _ in range(NBUF)]
            + [pltpu.VMEM((C, D), jnp.float32) for _ in range(NBUF)]
            + [pltpu.SemaphoreType.DMA for _ in range(3 * NBUF)]
        ),
    )(_sc_body)
    out = run(at, xi, yi,
              action_table.reshape(8 * D),
              x_table.reshape(64 * D),
              y_table.reshape(64 * D))
    return out.reshape(B, L, D)
